# baseline (device time: 16644 ns/iter reference)
import jax
import jax.numpy as jnp
from jax import lax
from jax.experimental import pallas as pl
from jax.experimental.pallas import tpu as pltpu

N_DEV = 4


def kernel(A, B):
    m, _ = A.shape
    _, n = B.shape
    q = m // N_DEV

    def body(a_ref, b_ref, out_ref, src_buf, rs_l, rs_r, rs_d,
             send_sems, recv_sems):
        my = lax.axis_index("i")
        left = (my - 1) % N_DEV
        right = (my + 1) % N_DEV
        diag = (my + 2) % N_DEV

        barrier_sem = pltpu.get_barrier_semaphore()
        for nbr in [left, right, diag]:
            pl.semaphore_signal(
                barrier_sem, inc=1,
                device_id=(nbr,), device_id_type=pl.DeviceIdType.MESH,
            )

        partial = jnp.dot(
            a_ref[:, :].astype(jnp.bfloat16),
            b_ref[:, :].astype(jnp.bfloat16),
            preferred_element_type=jnp.float32,
        )
        src_buf[:, :] = partial.astype(jnp.bfloat16)

        pl.semaphore_wait(barrier_sem, 3)

        rs_to_left = pltpu.make_async_remote_copy(
            src_ref=src_buf.at[pl.ds(left * q, q)], dst_ref=rs_r,
            send_sem=send_sems.at[0], recv_sem=recv_sems.at[1],
            device_id=(left,), device_id_type=pl.DeviceIdType.MESH,
        )
        rs_to_right = pltpu.make_async_remote_copy(
            src_ref=src_buf.at[pl.ds(right * q, q)], dst_ref=rs_l,
            send_sem=send_sems.at[1], recv_sem=recv_sems.at[0],
            device_id=(right,), device_id_type=pl.DeviceIdType.MESH,
        )
        rs_to_diag = pltpu.make_async_remote_copy(
            src_ref=src_buf.at[pl.ds(diag * q, q)], dst_ref=rs_d,
            send_sem=send_sems.at[2], recv_sem=recv_sems.at[2],
            device_id=(diag,), device_id_type=pl.DeviceIdType.MESH,
        )
        rs_to_left.start()
        rs_to_right.start()
        rs_to_diag.start()
        rs_to_left.wait_recv()
        rs_to_right.wait_recv()
        rs_to_diag.wait_recv()

        qsum = (
            src_buf[pl.ds(my * q, q), :].astype(jnp.float32)
            + rs_l[:, :].astype(jnp.float32)
            + rs_r[:, :].astype(jnp.float32)
            + rs_d[:, :].astype(jnp.float32)
        )
        s = qsum / (1.0 + jnp.exp(-qsum))
        out_ref[pl.ds(my * q, q), :] = s.astype(jnp.bfloat16)

        mine = out_ref.at[pl.ds(my * q, q)]
        ag_to_left = pltpu.make_async_remote_copy(
            src_ref=mine, dst_ref=mine,
            send_sem=send_sems.at[3], recv_sem=recv_sems.at[4],
            device_id=(left,), device_id_type=pl.DeviceIdType.MESH,
        )
        ag_to_right = pltpu.make_async_remote_copy(
            src_ref=mine, dst_ref=mine,
            send_sem=send_sems.at[4], recv_sem=recv_sems.at[3],
            device_id=(right,), device_id_type=pl.DeviceIdType.MESH,
        )
        ag_to_diag = pltpu.make_async_remote_copy(
            src_ref=mine, dst_ref=mine,
            send_sem=send_sems.at[5], recv_sem=recv_sems.at[5],
            device_id=(diag,), device_id_type=pl.DeviceIdType.MESH,
        )
        ag_to_left.start()
        ag_to_right.start()
        ag_to_diag.start()

        ag_from_left = pltpu.make_async_remote_copy(
            src_ref=out_ref.at[pl.ds(left * q, q)],
            dst_ref=out_ref.at[pl.ds(left * q, q)],
            send_sem=send_sems.at[3], recv_sem=recv_sems.at[3],
            device_id=(left,), device_id_type=pl.DeviceIdType.MESH,
        )
        ag_from_right = pltpu.make_async_remote_copy(
            src_ref=out_ref.at[pl.ds(right * q, q)],
            dst_ref=out_ref.at[pl.ds(right * q, q)],
            send_sem=send_sems.at[4], recv_sem=recv_sems.at[4],
            device_id=(right,), device_id_type=pl.DeviceIdType.MESH,
        )
        ag_from_diag = pltpu.make_async_remote_copy(
            src_ref=out_ref.at[pl.ds(diag * q, q)],
            dst_ref=out_ref.at[pl.ds(diag * q, q)],
            send_sem=send_sems.at[5], recv_sem=recv_sems.at[5],
            device_id=(diag,), device_id_type=pl.DeviceIdType.MESH,
        )
        ag_from_left.wait_recv()
        ag_from_right.wait_recv()
        ag_from_diag.wait_recv()

        rs_to_left.wait_send()
        rs_to_right.wait_send()
        rs_to_diag.wait_send()
        ag_to_left.wait_send()
        ag_to_right.wait_send()
        ag_to_diag.wait_send()

    return pl.pallas_call(
        body,
        out_shape=jax.ShapeDtypeStruct((m, n), jnp.bfloat16),
        in_specs=[
            pl.BlockSpec(memory_space=pltpu.VMEM),
            pl.BlockSpec(memory_space=pltpu.VMEM),
        ],
        out_specs=pl.BlockSpec(memory_space=pltpu.VMEM),
        scratch_shapes=[
            pltpu.VMEM((m, n), jnp.bfloat16),
            pltpu.VMEM((q, n), jnp.bfloat16),
            pltpu.VMEM((q, n), jnp.bfloat16),
            pltpu.VMEM((q, n), jnp.bfloat16),
            pltpu.SemaphoreType.DMA((6,)),
            pltpu.SemaphoreType.DMA((6,)),
        ],
        compiler_params=pltpu.CompilerParams(collective_id=0),
    )(A, B)


# device time: 14626 ns/iter; 1.1380x vs baseline; 1.1380x over previous
import jax
import jax.numpy as jnp
from jax import lax
from jax.experimental import pallas as pl
from jax.experimental.pallas import tpu as pltpu

N_DEV = 4
N_CHUNK = 2


def kernel(A, B):
    m, _ = A.shape
    _, n = B.shape
    q = m // N_DEV
    nc = n // N_CHUNK

    def body(a_ref, b_ref, out_ref, src_buf, rs_l, rs_r, rs_d,
             ag_src, ag_l, ag_r, ag_d, send_sems, recv_sems):
        my = lax.axis_index("i")
        left = (my - 1) % N_DEV
        right = (my + 1) % N_DEV
        diag = (my + 2) % N_DEV

        barrier_sem = pltpu.get_barrier_semaphore()
        for nbr in [left, right, diag]:
            pl.semaphore_signal(
                barrier_sem, inc=1,
                device_id=(nbr,), device_id_type=pl.DeviceIdType.MESH,
            )

        partial = jnp.dot(
            a_ref[:, :].astype(jnp.bfloat16),
            b_ref[:, :].astype(jnp.bfloat16),
            preferred_element_type=jnp.float32,
        )
        src_buf[:, :] = partial.astype(jnp.bfloat16)

        pl.semaphore_wait(barrier_sem, 3)

        def rs_chunk(c):
            to_left = pltpu.make_async_remote_copy(
                src_ref=src_buf.at[pl.ds(left * q, q), pl.ds(c * nc, nc)],
                dst_ref=rs_r.at[c],
                send_sem=send_sems.at[c * 6 + 0], recv_sem=recv_sems.at[c * 6 + 1],
                device_id=(left,), device_id_type=pl.DeviceIdType.MESH,
            )
            to_right = pltpu.make_async_remote_copy(
                src_ref=src_buf.at[pl.ds(right * q, q), pl.ds(c * nc, nc)],
                dst_ref=rs_l.at[c],
                send_sem=send_sems.at[c * 6 + 1], recv_sem=recv_sems.at[c * 6 + 0],
                device_id=(right,), device_id_type=pl.DeviceIdType.MESH,
            )
            to_diag = pltpu.make_async_remote_copy(
                src_ref=src_buf.at[pl.ds(diag * q, q), pl.ds(c * nc, nc)],
                dst_ref=rs_d.at[c],
                send_sem=send_sems.at[c * 6 + 2], recv_sem=recv_sems.at[c * 6 + 2],
                device_id=(diag,), device_id_type=pl.DeviceIdType.MESH,
            )
            return to_left, to_right, to_diag

        def ag_chunk(c):
            to_left = pltpu.make_async_remote_copy(
                src_ref=ag_src.at[c], dst_ref=ag_r.at[c],
                send_sem=send_sems.at[c * 6 + 3], recv_sem=recv_sems.at[c * 6 + 4],
                device_id=(left,), device_id_type=pl.DeviceIdType.MESH,
            )
            to_right = pltpu.make_async_remote_copy(
                src_ref=ag_src.at[c], dst_ref=ag_l.at[c],
                send_sem=send_sems.at[c * 6 + 4], recv_sem=recv_sems.at[c * 6 + 3],
                device_id=(right,), device_id_type=pl.DeviceIdType.MESH,
            )
            to_diag = pltpu.make_async_remote_copy(
                src_ref=ag_src.at[c], dst_ref=ag_d.at[c],
                send_sem=send_sems.at[c * 6 + 5], recv_sem=recv_sems.at[c * 6 + 5],
                device_id=(diag,), device_id_type=pl.DeviceIdType.MESH,
            )
            return to_left, to_right, to_diag

        rs = [rs_chunk(c) for c in range(N_CHUNK)]
        for c in range(N_CHUNK):
            for r in rs[c]:
                r.start()

        ag = []
        for c in range(N_CHUNK):
            for r in rs[c]:
                r.wait_recv()
            qsum = (
                src_buf[pl.ds(my * q, q), pl.ds(c * nc, nc)].astype(jnp.float32)
                + rs_l[c, :, :].astype(jnp.float32)
                + rs_r[c, :, :].astype(jnp.float32)
                + rs_d[c, :, :].astype(jnp.float32)
            )
            s = qsum / (1.0 + jnp.exp(-qsum))
            out_ref[pl.ds(my * q, q), pl.ds(c * nc, nc)] = s
            ag_src[c, :, :] = s.astype(jnp.bfloat16)
            ag.append(ag_chunk(c))
            for r in ag[c]:
                r.start()

        for c in range(N_CHUNK):
            a_left, a_right, a_diag = ag[c]
            a_left.wait_recv()
            out_ref[pl.ds(right * q, q), pl.ds(c * nc, nc)] = (
                ag_r[c, :, :].astype(jnp.float32))
            a_right.wait_recv()
            out_ref[pl.ds(left * q, q), pl.ds(c * nc, nc)] = (
                ag_l[c, :, :].astype(jnp.float32))
            a_diag.wait_recv()
            out_ref[pl.ds(diag * q, q), pl.ds(c * nc, nc)] = (
                ag_d[c, :, :].astype(jnp.float32))

        for c in range(N_CHUNK):
            for r in rs[c]:
                r.wait_send()
            for r in ag[c]:
                r.wait_send()

    return pl.pallas_call(
        body,
        out_shape=jax.ShapeDtypeStruct((m, n), jnp.float32),
        in_specs=[
            pl.BlockSpec(memory_space=pltpu.VMEM),
            pl.BlockSpec(memory_space=pltpu.VMEM),
        ],
        out_specs=pl.BlockSpec(memory_space=pltpu.VMEM),
        scratch_shapes=[
            pltpu.VMEM((m, n), jnp.bfloat16),
            pltpu.VMEM((N_CHUNK, q, nc), jnp.bfloat16),
            pltpu.VMEM((N_CHUNK, q, nc), jnp.bfloat16),
            pltpu.VMEM((N_CHUNK, q, nc), jnp.bfloat16),
            pltpu.VMEM((N_CHUNK, q, nc), jnp.bfloat16),
            pltpu.VMEM((N_CHUNK, q, nc), jnp.bfloat16),
            pltpu.VMEM((N_CHUNK, q, nc), jnp.bfloat16),
            pltpu.VMEM((N_CHUNK, q, nc), jnp.bfloat16),
            pltpu.SemaphoreType.DMA((6 * N_CHUNK,)),
            pltpu.SemaphoreType.DMA((6 * N_CHUNK,)),
        ],
        compiler_params=pltpu.CompilerParams(collective_id=0),
    )(A, B)
